# initial kernel scaffold (unmeasured)
import jax
import jax.numpy as jnp
from jax import lax
from jax.experimental import pallas as pl
from jax.experimental.pallas import tpu as pltpu

N_DEV = 4


def kernel(A, B):
    m, k = A.shape
    k2, n = B.shape
    m_chunk = m // N_DEV
    n_tile = 512
    n_tiles = n // n_tile

    def body(a_hbm, b_hbm, out_ref, comm_ref, a_vmem, b_vmem,
             send_sems, recv_sems, credit_sem, a_sem, b_sem):
        my = lax.axis_index("i")
        left = lax.rem(my + N_DEV - 1, N_DEV)
        right = lax.rem(my + 1, N_DEV)

        barrier_sem = pltpu.get_barrier_semaphore()
        for nbr in (left, right):
            pl.semaphore_signal(barrier_sem, inc=1, device_id=(nbr,),
                                device_id_type=pl.DeviceIdType.MESH)
        pl.semaphore_wait(barrier_sem, 2)

        def accum_chunk(c, slot, init):
            a_cp = pltpu.make_async_copy(
                a_hbm.at[pl.ds(c * m_chunk, m_chunk), :], a_vmem, a_sem)
            a_cp.start()
            a_cp.wait()
            for j in range(n_tiles):
                b_cp = pltpu.make_async_copy(
                    b_hbm.at[:, pl.ds(j * n_tile, n_tile)], b_vmem, b_sem)
                b_cp.start()
                b_cp.wait()
                p = jnp.dot(a_vmem[:, :], b_vmem[:, :],
                            preferred_element_type=jnp.float32)
                sl = pl.ds(j * n_tile, n_tile)
                if init:
                    comm_ref[slot, :, sl] = p
                else:
                    comm_ref[slot, :, sl] += p

        accum_chunk(lax.rem(my + N_DEV - 1, N_DEV), 0, True)

        for s in range(N_DEV - 1):
            send_slot = s % 2
            recv_slot = (s + 1) % 2
            rdma = pltpu.make_async_remote_copy(
                src_ref=comm_ref.at[send_slot],
                dst_ref=comm_ref.at[recv_slot],
                send_sem=send_sems.at[send_slot],
                recv_sem=recv_sems.at[recv_slot],
                device_id=(right,),
                device_id_type=pl.DeviceIdType.MESH,
            )
            if s >= 1:
                pl.semaphore_wait(credit_sem, 1)
            rdma.start()
            rdma.wait()
            if s <= 1:
                pl.semaphore_signal(credit_sem, inc=1, device_id=(left,),
                                    device_id_type=pl.DeviceIdType.MESH)
            c_recv = lax.rem(my + N_DEV + 2 - s, N_DEV)
            accum_chunk(c_recv, recv_slot, False)

        out_ref[:, :] = comm_ref[(N_DEV - 1) % 2, :, :]

    return pl.pallas_call(
        body,
        out_shape=jax.ShapeDtypeStruct((m_chunk, n), jnp.float32),
        in_specs=[pl.BlockSpec(memory_space=pltpu.ANY),
                  pl.BlockSpec(memory_space=pltpu.ANY)],
        out_specs=pl.BlockSpec(memory_space=pltpu.VMEM),
        scratch_shapes=[
            pltpu.VMEM((2, m_chunk, n), jnp.float32),
            pltpu.VMEM((m_chunk, k), jnp.float32),
            pltpu.VMEM((k, n_tile), jnp.float32),
            pltpu.SemaphoreType.DMA((2,)),
            pltpu.SemaphoreType.DMA((2,)),
            pltpu.SemaphoreType.REGULAR,
            pltpu.SemaphoreType.DMA,
            pltpu.SemaphoreType.DMA,
        ],
        compiler_params=pltpu.CompilerParams(collective_id=0),
    )(A, B)


# baseline (device time: 723379 ns/iter reference)
import jax
import jax.numpy as jnp
from jax import lax
from jax.experimental import pallas as pl
from jax.experimental.pallas import tpu as pltpu

N_DEV = 4


def kernel(A, B):
    m, k = A.shape
    k2, n = B.shape
    m_chunk = m // N_DEV
    n_tile = 512
    n_tiles = n // n_tile

    def body(a_hbm, b_hbm, out_ref, comm_ref, a_vmem, b_vmem,
             send_sems, recv_sems, credit_sem, a_sem, b_sem):
        my = lax.axis_index("i")
        left = lax.rem(my + N_DEV - 1, N_DEV)
        right = lax.rem(my + 1, N_DEV)

        barrier_sem = pltpu.get_barrier_semaphore()
        for nbr in (left, right):
            pl.semaphore_signal(barrier_sem, inc=1, device_id=(nbr,),
                                device_id_type=pl.DeviceIdType.MESH)
        pl.semaphore_wait(barrier_sem, 2)

        def accum_chunk(c, slot, init):
            a_cp = pltpu.make_async_copy(
                a_hbm.at[pl.ds(c * m_chunk, m_chunk), :], a_vmem, a_sem)
            a_cp.start()
            a_cp.wait()
            for j in range(n_tiles):
                b_cp = pltpu.make_async_copy(
                    b_hbm.at[:, pl.ds(j * n_tile, n_tile)], b_vmem, b_sem)
                b_cp.start()
                b_cp.wait()
                p = jnp.dot(a_vmem[:, :], b_vmem[:, :],
                            preferred_element_type=jnp.float32)
                sl = pl.ds(j * n_tile, n_tile)
                if init:
                    comm_ref[slot, :, sl] = p
                else:
                    comm_ref[slot, :, sl] += p

        accum_chunk(lax.rem(my + N_DEV - 1, N_DEV), 0, True)

        for s in range(N_DEV - 1):
            send_slot = s % 2
            recv_slot = (s + 1) % 2
            rdma = pltpu.make_async_remote_copy(
                src_ref=comm_ref.at[send_slot],
                dst_ref=comm_ref.at[recv_slot],
                send_sem=send_sems.at[send_slot],
                recv_sem=recv_sems.at[recv_slot],
                device_id=(right,),
                device_id_type=pl.DeviceIdType.MESH,
            )
            if s >= 1:
                pl.semaphore_wait(credit_sem, 1)
            rdma.start()
            rdma.wait()
            if s <= 1:
                pl.semaphore_signal(credit_sem, inc=1, device_id=(left,),
                                    device_id_type=pl.DeviceIdType.MESH)
            c_recv = lax.rem(my + N_DEV + 2 - s, N_DEV)
            accum_chunk(c_recv, recv_slot, False)

        out_ref[:, :] = comm_ref[(N_DEV - 1) % 2, :, :]

    return pl.pallas_call(
        body,
        out_shape=jax.ShapeDtypeStruct((m_chunk, n), jnp.float32),
        in_specs=[pl.BlockSpec(memory_space=pl.ANY),
                  pl.BlockSpec(memory_space=pl.ANY)],
        out_specs=pl.BlockSpec(memory_space=pltpu.VMEM),
        scratch_shapes=[
            pltpu.VMEM((2, m_chunk, n), jnp.float32),
            pltpu.VMEM((m_chunk, k), jnp.float32),
            pltpu.VMEM((k, n_tile), jnp.float32),
            pltpu.SemaphoreType.DMA((2,)),
            pltpu.SemaphoreType.DMA((2,)),
            pltpu.SemaphoreType.REGULAR,
            pltpu.SemaphoreType.DMA,
            pltpu.SemaphoreType.DMA,
        ],
        compiler_params=pltpu.CompilerParams(
            collective_id=0, vmem_limit_bytes=100 * 1024 * 1024),
    )(A, B)


# device time: 346914 ns/iter; 2.0852x vs baseline; 2.0852x over previous
import jax
import jax.numpy as jnp
from jax import lax
from jax.experimental import pallas as pl
from jax.experimental.pallas import tpu as pltpu

N_DEV = 4


def kernel(A, B):
    m, k = A.shape
    k2, n = B.shape
    m_chunk = m // N_DEV
    half_n = n // 2
    n_tile = 512
    half_tiles = half_n // n_tile

    def body(a_hbm, b_hbm, out_ref,
             comm_cw, comm_ccw, temp_cw, temp_ccw, a_vmem, b_vmem,
             send_cw, recv_cw, send_ccw, recv_ccw,
             credit_cw, credit_ccw, a_sem, b_sem, o_sem):
        my = lax.axis_index("i")
        left = lax.rem(my + N_DEV - 1, N_DEV)
        right = lax.rem(my + 1, N_DEV)

        barrier_sem = pltpu.get_barrier_semaphore()
        for nbr in (left, right):
            pl.semaphore_signal(barrier_sem, inc=1, device_id=(nbr,),
                                device_id_type=pl.DeviceIdType.MESH)
        pl.semaphore_wait(barrier_sem, 2)

        def load_a(c):
            cp = pltpu.make_async_copy(
                a_hbm.at[pl.ds(c * m_chunk, m_chunk), :], a_vmem, a_sem)
            cp.start()
            cp.wait()

        def compute_half(col_off, tgt):
            for j in range(half_tiles):
                cp = pltpu.make_async_copy(
                    b_hbm.at[:, pl.ds(col_off + j * n_tile, n_tile)],
                    b_vmem, b_sem)
                cp.start()
                cp.wait()
                tgt[:, pl.ds(j * n_tile, n_tile)] = jnp.dot(
                    a_vmem[:, :], b_vmem[:, :],
                    preferred_element_type=jnp.float32)

        load_a(lax.rem(my + N_DEV - 1, N_DEV))
        compute_half(0, comm_cw.at[0])
        load_a(lax.rem(my + 1, N_DEV))
        compute_half(half_n, comm_ccw.at[0])

        for s in range(N_DEV - 1):
            send_slot = s % 2
            recv_slot = (s + 1) % 2
            rdma_cw = pltpu.make_async_remote_copy(
                src_ref=comm_cw.at[send_slot],
                dst_ref=comm_cw.at[recv_slot],
                send_sem=send_cw.at[send_slot],
                recv_sem=recv_cw.at[recv_slot],
                device_id=(right,),
                device_id_type=pl.DeviceIdType.MESH,
            )
            rdma_ccw = pltpu.make_async_remote_copy(
                src_ref=comm_ccw.at[send_slot],
                dst_ref=comm_ccw.at[recv_slot],
                send_sem=send_ccw.at[send_slot],
                recv_sem=recv_ccw.at[recv_slot],
                device_id=(left,),
                device_id_type=pl.DeviceIdType.MESH,
            )
            if s >= 1:
                pl.semaphore_wait(credit_cw, 1)
                pl.semaphore_wait(credit_ccw, 1)
            rdma_cw.start()
            rdma_ccw.start()

            c_cw = lax.rem(my + N_DEV + 2 - s, N_DEV)
            c_ccw = lax.rem(my + 2 + s, N_DEV)
            load_a(c_cw)
            compute_half(0, temp_cw)
            if s % 2 == 1:
                load_a(c_ccw)
            compute_half(half_n, temp_ccw)

            rdma_cw.wait()
            rdma_ccw.wait()
            if s <= 1:
                pl.semaphore_signal(credit_cw, inc=1, device_id=(left,),
                                    device_id_type=pl.DeviceIdType.MESH)
                pl.semaphore_signal(credit_ccw, inc=1, device_id=(right,),
                                    device_id_type=pl.DeviceIdType.MESH)
            comm_cw[recv_slot, :, :] += temp_cw[:, :]
            comm_ccw[recv_slot, :, :] += temp_ccw[:, :]

        final_slot = (N_DEV - 1) % 2
        o1 = pltpu.make_async_copy(
            comm_cw.at[final_slot], out_ref.at[:, pl.ds(0, half_n)], o_sem)
        o1.start()
        o1.wait()
        o2 = pltpu.make_async_copy(
            comm_ccw.at[final_slot], out_ref.at[:, pl.ds(half_n, half_n)],
            o_sem)
        o2.start()
        o2.wait()

    return pl.pallas_call(
        body,
        out_shape=jax.ShapeDtypeStruct((m_chunk, n), jnp.float32),
        in_specs=[pl.BlockSpec(memory_space=pl.ANY),
                  pl.BlockSpec(memory_space=pl.ANY)],
        out_specs=pl.BlockSpec(memory_space=pl.ANY),
        scratch_shapes=[
            pltpu.VMEM((2, m_chunk, half_n), jnp.float32),
            pltpu.VMEM((2, m_chunk, half_n), jnp.float32),
            pltpu.VMEM((m_chunk, half_n), jnp.float32),
            pltpu.VMEM((m_chunk, half_n), jnp.float32),
            pltpu.VMEM((m_chunk, k), jnp.float32),
            pltpu.VMEM((k, n_tile), jnp.float32),
            pltpu.SemaphoreType.DMA((2,)),
            pltpu.SemaphoreType.DMA((2,)),
            pltpu.SemaphoreType.DMA((2,)),
            pltpu.SemaphoreType.DMA((2,)),
            pltpu.SemaphoreType.REGULAR,
            pltpu.SemaphoreType.REGULAR,
            pltpu.SemaphoreType.DMA,
            pltpu.SemaphoreType.DMA,
            pltpu.SemaphoreType.DMA,
        ],
        compiler_params=pltpu.CompilerParams(
            collective_id=0, vmem_limit_bytes=100 * 1024 * 1024),
    )(A, B)


# device time: 324999 ns/iter; 2.2258x vs baseline; 1.0674x over previous
import jax
import jax.numpy as jnp
from jax import lax
from jax.experimental import pallas as pl
from jax.experimental.pallas import tpu as pltpu

N_DEV = 4
N_STREAM = 4


def kernel(A, B):
    m, k = A.shape
    k2, n = B.shape
    m_chunk = m // N_DEV
    q_n = n // N_STREAM
    n_tile = 512
    q_tiles = q_n // n_tile

    OFFS = (2 * q_n, 3 * q_n, 0, q_n)
    DIRS = (-1, -1, +1, +1)

    def body(a_hbm, b_hbm, out_ref,
             c0, c1, c2, c3, t0, t1, t2, t3, a_vmem, b_vmem,
             sd0, sd1, sd2, sd3, rv0, rv1, rv2, rv3,
             cr0, cr1, cr2, cr3, a_sem, b_sem, o_sems):
        comms = (c0, c1, c2, c3)
        temps = (t0, t1, t2, t3)
        sends = (sd0, sd1, sd2, sd3)
        recvs = (rv0, rv1, rv2, rv3)
        credits = (cr0, cr1, cr2, cr3)

        my = lax.axis_index("i")
        left = lax.rem(my + N_DEV - 1, N_DEV)
        right = lax.rem(my + 1, N_DEV)
        tgt = {+1: right, -1: left}
        upstream = {+1: left, -1: right}

        barrier_sem = pltpu.get_barrier_semaphore()
        for nbr in (left, right):
            pl.semaphore_signal(barrier_sem, inc=1, device_id=(nbr,),
                                device_id_type=pl.DeviceIdType.MESH)
        pl.semaphore_wait(barrier_sem, 2)

        def load_a(c):
            cp = pltpu.make_async_copy(
                a_hbm.at[pl.ds(c * m_chunk, m_chunk), :], a_vmem, a_sem)
            cp.start()
            cp.wait()

        def compute_quarter(col_off, dst):
            for j in range(q_tiles):
                cp = pltpu.make_async_copy(
                    b_hbm.at[:, pl.ds(col_off + j * n_tile, n_tile)],
                    b_vmem, b_sem)
                cp.start()
                cp.wait()
                dst[:, pl.ds(j * n_tile, n_tile)] = jnp.dot(
                    a_vmem[:, :], b_vmem[:, :],
                    preferred_element_type=jnp.float32)

        def mk_rdma(kk, s):
            ss, rs = s % 2, (s + 1) % 2
            return pltpu.make_async_remote_copy(
                src_ref=comms[kk].at[ss],
                dst_ref=comms[kk].at[rs],
                send_sem=sends[kk].at[ss],
                recv_sem=recvs[kk].at[rs],
                device_id=(tgt[DIRS[kk]],),
                device_id_type=pl.DeviceIdType.MESH,
            )

        rd = [None] * N_STREAM
        ocp = [None] * N_STREAM

        load_a(lax.rem(my + 1, N_DEV))
        for kk in (0, 1):
            compute_quarter(OFFS[kk], comms[kk].at[0])
            rd[kk] = mk_rdma(kk, 0)
            rd[kk].start()
        load_a(lax.rem(my + N_DEV - 1, N_DEV))
        for kk in (2, 3):
            compute_quarter(OFFS[kk], comms[kk].at[0])
            rd[kk] = mk_rdma(kk, 0)
            rd[kk].start()

        for s in range(N_DEV - 1):
            recv_slot = (s + 1) % 2
            load_a(lax.rem(my + 2 + s, N_DEV))
            compute_quarter(OFFS[0], temps[0])
            compute_quarter(OFFS[1], temps[1])
            if s % 2 == 1:
                load_a(lax.rem(my + N_DEV + 2 - s, N_DEV))
            compute_quarter(OFFS[2], temps[2])
            compute_quarter(OFFS[3], temps[3])

            for kk in range(N_STREAM):
                rd[kk].wait()
                if s <= 1:
                    pl.semaphore_signal(
                        credits[kk], inc=1,
                        device_id=(upstream[DIRS[kk]],),
                        device_id_type=pl.DeviceIdType.MESH)
                comms[kk][recv_slot, :, :] += temps[kk][:, :]
                if s < N_DEV - 2:
                    pl.semaphore_wait(credits[kk], 1)
                    rd[kk] = mk_rdma(kk, s + 1)
                    rd[kk].start()
                else:
                    ocp[kk] = pltpu.make_async_copy(
                        comms[kk].at[recv_slot],
                        out_ref.at[:, pl.ds(OFFS[kk], q_n)],
                        o_sems.at[kk])
                    ocp[kk].start()

        for kk in range(N_STREAM):
            ocp[kk].wait()

    return pl.pallas_call(
        body,
        out_shape=jax.ShapeDtypeStruct((m_chunk, n), jnp.float32),
        in_specs=[pl.BlockSpec(memory_space=pl.ANY),
                  pl.BlockSpec(memory_space=pl.ANY)],
        out_specs=pl.BlockSpec(memory_space=pl.ANY),
        scratch_shapes=(
            [pltpu.VMEM((2, m_chunk, q_n), jnp.float32)] * N_STREAM
            + [pltpu.VMEM((m_chunk, q_n), jnp.float32)] * N_STREAM
            + [pltpu.VMEM((m_chunk, k), jnp.float32),
               pltpu.VMEM((k, n_tile), jnp.float32)]
            + [pltpu.SemaphoreType.DMA((2,))] * N_STREAM
            + [pltpu.SemaphoreType.DMA((2,))] * N_STREAM
            + [pltpu.SemaphoreType.REGULAR] * N_STREAM
            + [pltpu.SemaphoreType.DMA,
               pltpu.SemaphoreType.DMA,
               pltpu.SemaphoreType.DMA((N_STREAM,))]
        ),
        compiler_params=pltpu.CompilerParams(
            collective_id=0, vmem_limit_bytes=100 * 1024 * 1024),
    )(A, B)


# device time: 322975 ns/iter; 2.2397x vs baseline; 1.0063x over previous
import jax
import jax.numpy as jnp
from jax import lax
from jax.experimental import pallas as pl
from jax.experimental.pallas import tpu as pltpu

N_DEV = 4
N_STREAM = 4


def kernel(A, B):
    m, k = A.shape
    k2, n = B.shape
    m_chunk = m // N_DEV
    q_n = n // N_STREAM
    n_tile = 512
    q_tiles = q_n // n_tile

    OFFS = (2 * q_n, 3 * q_n, 0, q_n)
    DIRS = (-1, -1, +1, +1)

    def body(a_hbm, b_hbm, out_ref,
             c0, c1, c2, c3, t0, t1, t2, t3, a_vmem, b_vmem,
             sd0, sd1, sd2, sd3, rv0, rv1, rv2, rv3,
             cr0, cr1, cr2, cr3, a_sem, b_sem, o_sems):
        comms = (c0, c1, c2, c3)
        temps = (t0, t1, t2, t3)
        sends = (sd0, sd1, sd2, sd3)
        recvs = (rv0, rv1, rv2, rv3)
        credits = (cr0, cr1, cr2, cr3)

        my = lax.axis_index("i")
        left = lax.rem(my + N_DEV - 1, N_DEV)
        right = lax.rem(my + 1, N_DEV)
        tgt = {+1: right, -1: left}
        upstream = {+1: left, -1: right}

        barrier_sem = pltpu.get_barrier_semaphore()
        for nbr in (left, right):
            pl.semaphore_signal(barrier_sem, inc=1, device_id=(nbr,),
                                device_id_type=pl.DeviceIdType.MESH)
        pl.semaphore_wait(barrier_sem, 2)

        def load_a(c):
            cp = pltpu.make_async_copy(
                a_hbm.at[pl.ds(c * m_chunk, m_chunk), :], a_vmem, a_sem)
            cp.start()
            cp.wait()

        def compute_quarter(col_off, dst):
            for j in range(q_tiles):
                cp = pltpu.make_async_copy(
                    b_hbm.at[:, pl.ds(col_off + j * n_tile, n_tile)],
                    b_vmem, b_sem)
                cp.start()
                cp.wait()
                dst[:, pl.ds(j * n_tile, n_tile)] = jnp.dot(
                    a_vmem[:, :], b_vmem[:, :],
                    preferred_element_type=jnp.float32)

        def mk_rdma(kk, s):
            ss, rs = s % 2, (s + 1) % 2
            return pltpu.make_async_remote_copy(
                src_ref=comms[kk].at[ss],
                dst_ref=comms[kk].at[rs],
                send_sem=sends[kk].at[ss],
                recv_sem=recvs[kk].at[rs],
                device_id=(tgt[DIRS[kk]],),
                device_id_type=pl.DeviceIdType.MESH,
            )

        rd = [None] * N_STREAM
        ocp = [None] * N_STREAM

        for kk in (0, 2, 1, 3):
            if DIRS[kk] < 0:
                load_a(lax.rem(my + 1, N_DEV))
            else:
                load_a(lax.rem(my + N_DEV - 1, N_DEV))
            compute_quarter(OFFS[kk], comms[kk].at[0])
            rd[kk] = mk_rdma(kk, 0)
            rd[kk].start()

        for s in range(N_DEV - 1):
            recv_slot = (s + 1) % 2
            for kk in range(N_STREAM):
                if kk == 0:
                    load_a(lax.rem(my + 2 + s, N_DEV))
                elif kk == 2 and s % 2 == 1:
                    load_a(lax.rem(my + N_DEV + 2 - s, N_DEV))
                compute_quarter(OFFS[kk], temps[kk])

                rd[kk].wait()
                if s <= 1:
                    pl.semaphore_signal(
                        credits[kk], inc=1,
                        device_id=(upstream[DIRS[kk]],),
                        device_id_type=pl.DeviceIdType.MESH)
                comms[kk][recv_slot, :, :] += temps[kk][:, :]
                if s < N_DEV - 2:
                    pl.semaphore_wait(credits[kk], 1)
                    rd[kk] = mk_rdma(kk, s + 1)
                    rd[kk].start()
                else:
                    ocp[kk] = pltpu.make_async_copy(
                        comms[kk].at[recv_slot],
                        out_ref.at[:, pl.ds(OFFS[kk], q_n)],
                        o_sems.at[kk])
                    ocp[kk].start()

        for kk in range(N_STREAM):
            ocp[kk].wait()

    return pl.pallas_call(
        body,
        out_shape=jax.ShapeDtypeStruct((m_chunk, n), jnp.float32),
        in_specs=[pl.BlockSpec(memory_space=pl.ANY),
                  pl.BlockSpec(memory_space=pl.ANY)],
        out_specs=pl.BlockSpec(memory_space=pl.ANY),
        scratch_shapes=(
            [pltpu.VMEM((2, m_chunk, q_n), jnp.float32)] * N_STREAM
            + [pltpu.VMEM((m_chunk, q_n), jnp.float32)] * N_STREAM
            + [pltpu.VMEM((m_chunk, k), jnp.float32),
               pltpu.VMEM((k, n_tile), jnp.float32)]
            + [pltpu.SemaphoreType.DMA((2,))] * N_STREAM
            + [pltpu.SemaphoreType.DMA((2,))] * N_STREAM
            + [pltpu.SemaphoreType.REGULAR] * N_STREAM
            + [pltpu.SemaphoreType.DMA,
               pltpu.SemaphoreType.DMA,
               pltpu.SemaphoreType.DMA((N_STREAM,))]
        ),
        compiler_params=pltpu.CompilerParams(
            collective_id=0, vmem_limit_bytes=100 * 1024 * 1024),
    )(A, B)


# device time: 182385 ns/iter; 3.9662x vs baseline; 1.7708x over previous
import jax
import jax.numpy as jnp
from jax import lax
from jax.experimental import pallas as pl
from jax.experimental.pallas import tpu as pltpu

N_DEV = 4
N_STREAM = 4


def kernel(A, B):
    m, k = A.shape
    k2, n = B.shape
    m_chunk = m // N_DEV
    q_n = n // N_STREAM
    n_tile = 512
    q_tiles = q_n // n_tile

    OFFS = (2 * q_n, 3 * q_n, 0, q_n)
    DIRS = (-1, -1, +1, +1)

    def body(a_hbm, b_hbm, out_ref,
             c0, c1, c2, c3, t0, t1, t2, t3, a_vmem, b_vmem,
             sd0, sd1, sd2, sd3, rv0, rv1, rv2, rv3,
             cr0, cr1, cr2, cr3, a_sem, b_sems, o_sems):
        comms = (c0, c1, c2, c3)
        temps = (t0, t1, t2, t3)
        sends = (sd0, sd1, sd2, sd3)
        recvs = (rv0, rv1, rv2, rv3)
        credits = (cr0, cr1, cr2, cr3)

        my = lax.axis_index("i")
        left = lax.rem(my + N_DEV - 1, N_DEV)
        right = lax.rem(my + 1, N_DEV)
        tgt = {+1: right, -1: left}
        upstream = {+1: left, -1: right}

        barrier_sem = pltpu.get_barrier_semaphore()
        for nbr in (left, right):
            pl.semaphore_signal(barrier_sem, inc=1, device_id=(nbr,),
                                device_id_type=pl.DeviceIdType.MESH)
        pl.semaphore_wait(barrier_sem, 2)

        def load_a(c):
            cp = pltpu.make_async_copy(
                a_hbm.at[pl.ds(c * m_chunk, m_chunk), :], a_vmem, a_sem)
            cp.start()
            cp.wait()

        def b_tile_copy(col_off, j):
            return pltpu.make_async_copy(
                b_hbm.at[:, pl.ds(col_off + j * n_tile, n_tile)],
                b_vmem.at[j % 2], b_sems.at[j % 2])

        def compute_quarter(col_off, dst):
            b_tile_copy(col_off, 0).start()
            for j in range(q_tiles):
                if j + 1 < q_tiles:
                    b_tile_copy(col_off, j + 1).start()
                b_tile_copy(col_off, j).wait()
                dst[:, pl.ds(j * n_tile, n_tile)] = jnp.dot(
                    a_vmem[:, :], b_vmem[j % 2, :, :],
                    preferred_element_type=jnp.float32)

        def mk_rdma(kk, s):
            ss, rs = s % 2, (s + 1) % 2
            return pltpu.make_async_remote_copy(
                src_ref=comms[kk].at[ss],
                dst_ref=comms[kk].at[rs],
                send_sem=sends[kk].at[ss],
                recv_sem=recvs[kk].at[rs],
                device_id=(tgt[DIRS[kk]],),
                device_id_type=pl.DeviceIdType.MESH,
            )

        rd = [None] * N_STREAM
        ocp = [None] * N_STREAM

        for kk in (0, 2, 1, 3):
            if DIRS[kk] < 0:
                load_a(lax.rem(my + 1, N_DEV))
            else:
                load_a(lax.rem(my + N_DEV - 1, N_DEV))
            compute_quarter(OFFS[kk], temps[kk])
            comms[kk][0, :, :] = temps[kk][:, :].astype(jnp.bfloat16)
            rd[kk] = mk_rdma(kk, 0)
            rd[kk].start()

        for s in range(N_DEV - 1):
            recv_slot = (s + 1) % 2
            for kk in range(N_STREAM):
                if kk == 0:
                    load_a(lax.rem(my + 2 + s, N_DEV))
                elif kk == 2 and s % 2 == 1:
                    load_a(lax.rem(my + N_DEV + 2 - s, N_DEV))
                compute_quarter(OFFS[kk], temps[kk])

                rd[kk].wait()
                if s <= 1:
                    pl.semaphore_signal(
                        credits[kk], inc=1,
                        device_id=(upstream[DIRS[kk]],),
                        device_id_type=pl.DeviceIdType.MESH)
                if s < N_DEV - 2:
                    comms[kk][recv_slot, :, :] = (
                        comms[kk][recv_slot, :, :].astype(jnp.float32)
                        + temps[kk][:, :]).astype(jnp.bfloat16)
                    pl.semaphore_wait(credits[kk], 1)
                    rd[kk] = mk_rdma(kk, s + 1)
                    rd[kk].start()
                else:
                    temps[kk][:, :] += comms[kk][
                        recv_slot, :, :].astype(jnp.float32)
                    ocp[kk] = pltpu.make_async_copy(
                        temps[kk],
                        out_ref.at[:, pl.ds(OFFS[kk], q_n)],
                        o_sems.at[kk])
                    ocp[kk].start()

        for kk in range(N_STREAM):
            ocp[kk].wait()

    return pl.pallas_call(
        body,
        out_shape=jax.ShapeDtypeStruct((m_chunk, n), jnp.float32),
        in_specs=[pl.BlockSpec(memory_space=pl.ANY),
                  pl.BlockSpec(memory_space=pl.ANY)],
        out_specs=pl.BlockSpec(memory_space=pl.ANY),
        scratch_shapes=(
            [pltpu.VMEM((2, m_chunk, q_n), jnp.bfloat16)] * N_STREAM
            + [pltpu.VMEM((m_chunk, q_n), jnp.float32)] * N_STREAM
            + [pltpu.VMEM((m_chunk, k), jnp.float32),
               pltpu.VMEM((2, k, n_tile), jnp.float32)]
            + [pltpu.SemaphoreType.DMA((2,))] * N_STREAM
            + [pltpu.SemaphoreType.DMA((2,))] * N_STREAM
            + [pltpu.SemaphoreType.REGULAR] * N_STREAM
            + [pltpu.SemaphoreType.DMA,
               pltpu.SemaphoreType.DMA((2,)),
               pltpu.SemaphoreType.DMA((N_STREAM,))]
        ),
        compiler_params=pltpu.CompilerParams(
            collective_id=0, vmem_limit_bytes=100 * 1024 * 1024),
    )(A, B)


# device time: 182219 ns/iter; 3.9698x vs baseline; 1.0009x over previous
import jax
import jax.numpy as jnp
from jax import lax
from jax.experimental import pallas as pl
from jax.experimental.pallas import tpu as pltpu

N_DEV = 4
N_STREAM = 4


def kernel(A, B):
    m, k = A.shape
    k2, n = B.shape
    m_chunk = m // N_DEV
    q_n = n // N_STREAM
    n_tile = 512
    q_tiles = q_n // n_tile

    OFFS = (2 * q_n, 3 * q_n, 0, q_n)
    DIRS = (-1, -1, +1, +1)

    def body(a_hbm, b_hbm, out_ref,
             c0, c1, c2, c3, t0, t1, t2, t3, a_vmem, b_vmem,
             sd0, sd1, sd2, sd3, rv0, rv1, rv2, rv3,
             cr0, cr1, cr2, cr3, a_sem, b_sems, o_sems):
        comms = (c0, c1, c2, c3)
        temps = (t0, t1, t2, t3)
        sends = (sd0, sd1, sd2, sd3)
        recvs = (rv0, rv1, rv2, rv3)
        credits = (cr0, cr1, cr2, cr3)

        my = lax.axis_index("i")
        left = lax.rem(my + N_DEV - 1, N_DEV)
        right = lax.rem(my + 1, N_DEV)
        tgt = {+1: right, -1: left}
        upstream = {+1: left, -1: right}

        barrier_sem = pltpu.get_barrier_semaphore()
        for nbr in (left, right):
            pl.semaphore_signal(barrier_sem, inc=1, device_id=(nbr,),
                                device_id_type=pl.DeviceIdType.MESH)
        pl.semaphore_wait(barrier_sem, 2)

        def load_a(c):
            cp = pltpu.make_async_copy(
                a_hbm.at[pl.ds(c * m_chunk, m_chunk), :], a_vmem, a_sem)
            cp.start()
            cp.wait()

        def b_tile_copy(col_off, j):
            return pltpu.make_async_copy(
                b_hbm.at[:, pl.ds(col_off + j * n_tile, n_tile)],
                b_vmem.at[j % 2], b_sems.at[j % 2])

        def compute_quarter(col_off, dst, to_bf16=False):
            b_tile_copy(col_off, 0).start()
            for j in range(q_tiles):
                if j + 1 < q_tiles:
                    b_tile_copy(col_off, j + 1).start()
                b_tile_copy(col_off, j).wait()
                p = jnp.dot(a_vmem[:, :], b_vmem[j % 2, :, :],
                            preferred_element_type=jnp.float32)
                if to_bf16:
                    p = p.astype(jnp.bfloat16)
                dst[:, pl.ds(j * n_tile, n_tile)] = p

        def mk_rdma(kk, s):
            ss, rs = s % 2, (s + 1) % 2
            return pltpu.make_async_remote_copy(
                src_ref=comms[kk].at[ss],
                dst_ref=comms[kk].at[rs],
                send_sem=sends[kk].at[ss],
                recv_sem=recvs[kk].at[rs],
                device_id=(tgt[DIRS[kk]],),
                device_id_type=pl.DeviceIdType.MESH,
            )

        rd = [None] * N_STREAM
        ocp = [None] * N_STREAM

        for kk in (0, 2, 1, 3):
            if DIRS[kk] < 0:
                load_a(lax.rem(my + 1, N_DEV))
            else:
                load_a(lax.rem(my + N_DEV - 1, N_DEV))
            compute_quarter(OFFS[kk], comms[kk].at[0], to_bf16=True)
            rd[kk] = mk_rdma(kk, 0)
            rd[kk].start()

        for s in range(N_DEV - 1):
            recv_slot = (s + 1) % 2
            for kk in range(N_STREAM):
                if kk == 0:
                    load_a(lax.rem(my + 2 + s, N_DEV))
                elif kk == 2 and s % 2 == 1:
                    load_a(lax.rem(my + N_DEV + 2 - s, N_DEV))
                compute_quarter(OFFS[kk], temps[kk])

                rd[kk].wait()
                if s <= 1:
                    pl.semaphore_signal(
                        credits[kk], inc=1,
                        device_id=(upstream[DIRS[kk]],),
                        device_id_type=pl.DeviceIdType.MESH)
                if s < N_DEV - 2:
                    comms[kk][recv_slot, :, :] = (
                        comms[kk][recv_slot, :, :].astype(jnp.float32)
                        + temps[kk][:, :]).astype(jnp.bfloat16)
                    pl.semaphore_wait(credits[kk], 1)
                    rd[kk] = mk_rdma(kk, s + 1)
                    rd[kk].start()
                else:
                    temps[kk][:, :] += comms[kk][
                        recv_slot, :, :].astype(jnp.float32)
                    ocp[kk] = pltpu.make_async_copy(
                        temps[kk],
                        out_ref.at[:, pl.ds(OFFS[kk], q_n)],
                        o_sems.at[kk])
                    ocp[kk].start()

        for kk in range(N_STREAM):
            ocp[kk].wait()

    return pl.pallas_call(
        body,
        out_shape=jax.ShapeDtypeStruct((m_chunk, n), jnp.float32),
        in_specs=[pl.BlockSpec(memory_space=pl.ANY),
                  pl.BlockSpec(memory_space=pl.ANY)],
        out_specs=pl.BlockSpec(memory_space=pl.ANY),
        scratch_shapes=(
            [pltpu.VMEM((2, m_chunk, q_n), jnp.bfloat16)] * N_STREAM
            + [pltpu.VMEM((m_chunk, q_n), jnp.float32)] * N_STREAM
            + [pltpu.VMEM((m_chunk, k), jnp.float32),
               pltpu.VMEM((2, k, n_tile), jnp.float32)]
            + [pltpu.SemaphoreType.DMA((2,))] * N_STREAM
            + [pltpu.SemaphoreType.DMA((2,))] * N_STREAM
            + [pltpu.SemaphoreType.REGULAR] * N_STREAM
            + [pltpu.SemaphoreType.DMA,
               pltpu.SemaphoreType.DMA((2,)),
               pltpu.SemaphoreType.DMA((N_STREAM,))]
        ),
        compiler_params=pltpu.CompilerParams(
            collective_id=0, vmem_limit_bytes=100 * 1024 * 1024),
    )(A, B)
